# Initial kernel scaffold; baseline (speedup 1.0000x reference)
import jax
import jax.numpy as jnp
from jax.experimental import pallas as pl


def kernel(x, edge_index, W1, b1, W2, b2):
    raise NotImplementedError("write your pallas kernel here")


# speedup vs baseline: 12.0522x; 12.0522x over previous
import functools

import jax
import jax.numpy as jnp
from jax import lax
from jax.experimental import pallas as pl
from jax.experimental.pallas import tpu as pltpu
from jax.experimental.pallas import tpu_sc as plsc

N = 10000
E = 320000
D_IN = 128
D_H = 256
D_LAT = 128

NPAD = 10240
ROWB = 1024
GRID = NPAD // ROWB

NSUB = 16
NCORE = 2
CH = 128
NCH1 = 158
NCH2 = 80
ROWS_PER_SUB = NPAD // NSUB
DEG_W = NCORE * NSUB
DEG_PER_W = E // DEG_W

_SC_PARAMS = pltpu.CompilerParams(needs_layout_passes=False)


def _make_sc_deg(mesh):
    @functools.partial(
        pl.kernel,
        mesh=mesh,
        out_type=jax.ShapeDtypeStruct((DEG_W, NPAD), jnp.float32),
        compiler_params=_SC_PARAMS,
        scratch_types=[
            pltpu.VMEM((DEG_PER_W,), jnp.int32),
            pltpu.VMEM((NPAD,), jnp.float32),
        ],
    )
    def sc_deg(dst_hbm, out_hbm, dstv, acc):
        c = lax.axis_index("c")
        s = lax.axis_index("s")
        w = s * NCORE + c
        pltpu.sync_copy(dst_hbm.at[w], dstv)

        def zero_body(i, carry):
            acc[pl.ds(i * 16, 16)] = jnp.zeros((16,), jnp.float32)
            return carry

        lax.fori_loop(0, NPAD // 16, zero_body, None)

        ones = jnp.full((16,), 1.0, jnp.float32)

        def body(i, carry):
            idx = dstv[pl.ds(i * 16, 16)]
            plsc.addupdate_scatter(acc, [idx], ones)
            return carry

        lax.fori_loop(0, DEG_PER_W // 16, body, None)
        pltpu.sync_copy(acc, out_hbm.at[w])

    return sc_deg


def _agg_pipeline(idxp, g, tab, acc, ibufs, gbufs, isems, gsems, nch):
    for b in range(2):
        pltpu.async_copy(idxp.at[g, b], ibufs[b], isems[b])
    pltpu.make_async_copy(idxp.at[g, 0], ibufs[0], isems[0]).wait()
    pltpu.async_copy(tab.at[ibufs[0].at[0]], gbufs[0], gsems[0])

    def body(i, carry):
        for b in range(2):
            cc = 2 * i + b
            nb = 1 - b

            @pl.when(cc + 1 < nch)
            def _():
                pltpu.make_async_copy(idxp.at[g, cc + 1], ibufs[nb],
                                      isems[nb]).wait()
                pltpu.async_copy(tab.at[ibufs[nb].at[0]], gbufs[nb],
                                 gsems[nb])

            pltpu.make_async_copy(tab.at[ibufs[b].at[0]], gbufs[b],
                                  gsems[b]).wait()
            pltpu.sync_copy(gbufs[b], acc.at[ibufs[b].at[1]], add=True)

            @pl.when(cc + 2 < nch)
            def _():
                pltpu.async_copy(idxp.at[g, cc + 2], ibufs[b], isems[b])

        return carry

    lax.fori_loop(0, nch // 2, body, None)


def _agg_scratch():
    return [
        pltpu.VMEM((2, CH), jnp.int32),
        pltpu.VMEM((2, CH), jnp.int32),
        pltpu.VMEM((CH, D_IN), jnp.float32),
        pltpu.VMEM((CH, D_IN), jnp.float32),
        pltpu.VMEM_SHARED((NPAD, D_IN), jnp.float32),
        pltpu.SemaphoreType.DMA,
        pltpu.SemaphoreType.DMA,
        pltpu.SemaphoreType.DMA,
        pltpu.SemaphoreType.DMA,
    ]


def _make_sc_agg1(mesh):
    @functools.partial(
        pl.kernel,
        mesh=mesh,
        out_type=[
            jax.ShapeDtypeStruct((NPAD, D_IN), jnp.float32),
            jax.ShapeDtypeStruct((NPAD, D_IN), jnp.float32),
        ],
        compiler_params=_SC_PARAMS,
        scratch_types=_agg_scratch(),
    )
    def sc_agg1(idxp_hbm, taba_hbm, tabb_hbm, zeros_hbm,
                outa_hbm, outb_hbm,
                ibuf0, ibuf1, gbuf0, gbuf1, acc, is0, is1, gs0, gs1):
        c = lax.axis_index("c")
        s = lax.axis_index("s")
        row0 = s * ROWS_PER_SUB
        ibufs, gbufs = (ibuf0, ibuf1), (gbuf0, gbuf1)
        isems, gsems = (is0, is1), (gs0, gs1)

        pltpu.sync_copy(zeros_hbm.at[pl.ds(0, ROWS_PER_SUB)],
                        acc.at[pl.ds(row0, ROWS_PER_SUB)])
        plsc.subcore_barrier()

        def run(tab, outh):
            _agg_pipeline(idxp_hbm, s, tab, acc, ibufs, gbufs, isems,
                          gsems, NCH1)
            plsc.subcore_barrier()
            pltpu.sync_copy(acc.at[pl.ds(row0, ROWS_PER_SUB)],
                            outh.at[pl.ds(row0, ROWS_PER_SUB)])

        @pl.when(c == 0)
        def _():
            run(taba_hbm, outa_hbm)

        @pl.when(c == 1)
        def _():
            run(tabb_hbm, outb_hbm)

    return sc_agg1


def _make_sc_agg2(mesh):
    @functools.partial(
        pl.kernel,
        mesh=mesh,
        out_type=[
            jax.ShapeDtypeStruct((NPAD, D_IN), jnp.float32),
            jax.ShapeDtypeStruct((NPAD, D_IN), jnp.float32),
        ],
        compiler_params=_SC_PARAMS,
        scratch_types=_agg_scratch(),
    )
    def sc_agg2(idxp_hbm, tab_hbm, zeros_hbm, outa_hbm, outb_hbm,
                ibuf0, ibuf1, gbuf0, gbuf1, acc, is0, is1, gs0, gs1):
        c = lax.axis_index("c")
        s = lax.axis_index("s")
        w = s * NCORE + c
        row0 = s * ROWS_PER_SUB
        ibufs, gbufs = (ibuf0, ibuf1), (gbuf0, gbuf1)
        isems, gsems = (is0, is1), (gs0, gs1)

        pltpu.sync_copy(zeros_hbm.at[pl.ds(0, ROWS_PER_SUB)],
                        acc.at[pl.ds(row0, ROWS_PER_SUB)])
        plsc.subcore_barrier()

        _agg_pipeline(idxp_hbm, w, tab_hbm, acc, ibufs, gbufs, isems,
                      gsems, NCH2)
        plsc.subcore_barrier()

        @pl.when(c == 0)
        def _():
            pltpu.sync_copy(acc.at[pl.ds(row0, ROWS_PER_SUB)],
                            outa_hbm.at[pl.ds(row0, ROWS_PER_SUB)])

        @pl.when(c == 1)
        def _():
            pltpu.sync_copy(acc.at[pl.ds(row0, ROWS_PER_SUB)],
                            outb_hbm.at[pl.ds(row0, ROWS_PER_SUB)])

    return sc_agg2


_SC_CACHE = {}


def _sc_kernels():
    if not _SC_CACHE:
        mesh = plsc.VectorSubcoreMesh(core_axis_name="c", subcore_axis_name="s",
                                      num_cores=NCORE, num_subcores=NSUB)
        _SC_CACHE["deg"] = _make_sc_deg(mesh)
        _SC_CACHE["agg1"] = _make_sc_agg1(mesh)
        _SC_CACHE["agg2"] = _make_sc_agg2(mesh)
    return _SC_CACHE


def _dinv_of(degp):
    return lax.rsqrt(jnp.sum(degp, axis=0) + 1.0)


def _tc1_body(x_ref, w1_ref, degp_ref, ha_ref, hb_ref):
    dinv = _dinv_of(degp_ref[...])
    h = jnp.dot(x_ref[...], w1_ref[...], preferred_element_type=jnp.float32)
    hs = h * dinv[:, None]
    ha_ref[...] = hs[:, : D_H // 2]
    hb_ref[...] = hs[:, D_H // 2:]


def _tc2_body(a1a_ref, a1b_ref, h1a_ref, h1b_ref, degp_ref, b1_ref, w2_ref,
              o_ref):
    dinv = _dinv_of(degp_ref[...])
    b1 = b1_ref[...]
    za = jax.nn.relu((a1a_ref[...] + h1a_ref[...]) * dinv[:, None]
                     + b1[None, : D_H // 2])
    zb = jax.nn.relu((a1b_ref[...] + h1b_ref[...]) * dinv[:, None]
                     + b1[None, D_H // 2:])
    h2 = (jnp.dot(za, w2_ref[: D_H // 2, :],
                  preferred_element_type=jnp.float32)
          + jnp.dot(zb, w2_ref[D_H // 2:, :],
                    preferred_element_type=jnp.float32))
    o_ref[...] = h2 * dinv[:, None]


def _tc3_body(a2p0_ref, a2p1_ref, h2s_ref, degp_ref, b2_ref, o_ref):
    dinv = _dinv_of(degp_ref[...])
    o_ref[...] = ((a2p0_ref[...] + a2p1_ref[...] + h2s_ref[...])
                  * dinv[:, None] + b2_ref[...][None, :])


def _rows(shape_cols):
    return pl.BlockSpec((ROWB, shape_cols), lambda i: (i, 0))


def _full(shape):
    return pl.BlockSpec(shape, lambda i: tuple(0 for _ in shape))


_DEGP_SPEC = pl.BlockSpec((DEG_W, ROWB), lambda i: (0, i))

_tc1 = pl.pallas_call(
    _tc1_body,
    grid=(GRID,),
    in_specs=[_rows(D_IN), _full((D_IN, D_H)), _DEGP_SPEC],
    out_specs=[_rows(D_H // 2), _rows(D_H // 2)],
    out_shape=[
        jax.ShapeDtypeStruct((NPAD, D_H // 2), jnp.float32),
        jax.ShapeDtypeStruct((NPAD, D_H // 2), jnp.float32),
    ],
)

_tc2 = pl.pallas_call(
    _tc2_body,
    grid=(GRID,),
    in_specs=[_rows(D_H // 2)] * 4 + [_DEGP_SPEC, _full((D_H,)),
                                      _full((D_H, D_LAT))],
    out_specs=_rows(D_LAT),
    out_shape=jax.ShapeDtypeStruct((NPAD, D_LAT), jnp.float32),
)

_tc3 = pl.pallas_call(
    _tc3_body,
    grid=(GRID,),
    in_specs=[_rows(D_LAT), _rows(D_LAT), _rows(D_LAT), _DEGP_SPEC,
              _full((D_LAT,))],
    out_specs=_rows(D_LAT),
    out_shape=jax.ShapeDtypeStruct((NPAD, D_LAT), jnp.float32),
)


def _pack_idx(src, dst, groups, nch):
    pad = groups * nch * CH - E
    srcp = jnp.concatenate([src, jnp.zeros((pad,), jnp.int32)])
    dstp = jnp.concatenate([dst, jnp.full((pad,), N, jnp.int32)])
    return jnp.stack([srcp.reshape(groups, nch, CH),
                      dstp.reshape(groups, nch, CH)], axis=2)


def kernel(x, edge_index, W1, b1, W2, b2):
    src = edge_index[0]
    dst = edge_index[1]

    sc = _sc_kernels()

    degp = sc["deg"](dst.reshape(DEG_W, DEG_PER_W))

    idx1 = _pack_idx(src, dst, NSUB, NCH1)
    idx2 = _pack_idx(src, dst, DEG_W, NCH2)

    xp = jnp.zeros((NPAD, D_IN), jnp.float32).at[:N].set(x)
    zeros_acc = jnp.zeros((NPAD, D_IN), jnp.float32)

    h1a, h1b = _tc1(xp, W1, degp)
    a1a, a1b = sc["agg1"](idx1, h1a, h1b, zeros_acc)
    h2s = _tc2(a1a, a1b, h1a, h1b, degp, b1, W2)
    a2p0, a2p1 = sc["agg2"](idx2, h2s, zeros_acc)
    out = _tc3(a2p0, a2p1, h2s, degp, b2)
    return out[:N]


# speedup vs baseline: 12.0589x; 1.0006x over previous
import functools

import jax
import jax.numpy as jnp
from jax import lax
from jax.experimental import pallas as pl
from jax.experimental.pallas import tpu as pltpu
from jax.experimental.pallas import tpu_sc as plsc

N = 10000
E = 320000
D_IN = 128
D_H = 256
D_LAT = 128

NPAD = 10240
ROWB = 1024
GRID = NPAD // ROWB

NSUB = 16
NCORE = 2
CH = 128
NCH1 = 158
NCH2 = 80
ROWS_PER_SUB = NPAD // NSUB
DEG_W = NCORE * NSUB
DEG_PER_W = E // DEG_W

_SC_PARAMS = pltpu.CompilerParams(needs_layout_passes=False)


def _make_sc_deg(mesh):
    @functools.partial(
        pl.kernel,
        mesh=mesh,
        out_type=jax.ShapeDtypeStruct((DEG_W, NPAD), jnp.float32),
        compiler_params=_SC_PARAMS,
        scratch_types=[
            pltpu.VMEM((DEG_PER_W,), jnp.int32),
            pltpu.VMEM((NPAD,), jnp.float32),
        ],
    )
    def sc_deg(dst_hbm, out_hbm, dstv, acc):
        c = lax.axis_index("c")
        s = lax.axis_index("s")
        w = s * NCORE + c
        pltpu.sync_copy(dst_hbm.at[w], dstv)

        def zero_body(i, carry):
            acc[pl.ds(i * 16, 16)] = jnp.zeros((16,), jnp.float32)
            return carry

        lax.fori_loop(0, NPAD // 16, zero_body, None)

        ones = jnp.full((16,), 1.0, jnp.float32)

        def body(i, carry):
            idx = dstv[pl.ds(i * 16, 16)]
            plsc.addupdate_scatter(acc, [idx], ones)
            return carry

        lax.fori_loop(0, DEG_PER_W // 16, body, None)
        pltpu.sync_copy(acc, out_hbm.at[w])

    return sc_deg


def _agg_pipeline(idxp, g, tab, acc, ibufs, gbufs, isems, gsems, nch):
    for b in range(2):
        pltpu.async_copy(idxp.at[g, b], ibufs[b], isems[b])
    pltpu.make_async_copy(idxp.at[g, 0], ibufs[0], isems[0]).wait()
    pltpu.async_copy(tab.at[ibufs[0].at[0]], gbufs[0], gsems[0])

    def body(i, carry):
        for b in range(2):
            cc = 2 * i + b
            nb = 1 - b

            @pl.when(cc + 1 < nch)
            def _():
                pltpu.make_async_copy(idxp.at[g, cc + 1], ibufs[nb],
                                      isems[nb]).wait()
                pltpu.async_copy(tab.at[ibufs[nb].at[0]], gbufs[nb],
                                 gsems[nb])

            pltpu.make_async_copy(tab.at[ibufs[b].at[0]], gbufs[b],
                                  gsems[b]).wait()
            pltpu.sync_copy(gbufs[b], acc.at[ibufs[b].at[1]], add=True)

            @pl.when(cc + 2 < nch)
            def _():
                pltpu.async_copy(idxp.at[g, cc + 2], ibufs[b], isems[b])

        return carry

    lax.fori_loop(0, nch // 2, body, None)


def _agg_scratch():
    return [
        pltpu.VMEM((2, CH), jnp.int32),
        pltpu.VMEM((2, CH), jnp.int32),
        pltpu.VMEM((CH, D_IN), jnp.float32),
        pltpu.VMEM((CH, D_IN), jnp.float32),
        pltpu.VMEM_SHARED((NPAD, D_IN), jnp.float32),
        pltpu.SemaphoreType.DMA,
        pltpu.SemaphoreType.DMA,
        pltpu.SemaphoreType.DMA,
        pltpu.SemaphoreType.DMA,
    ]


def _make_sc_agg1(mesh):
    @functools.partial(
        pl.kernel,
        mesh=mesh,
        out_type=[
            jax.ShapeDtypeStruct((NPAD, D_IN), jnp.float32),
            jax.ShapeDtypeStruct((NPAD, D_IN), jnp.float32),
        ],
        compiler_params=_SC_PARAMS,
        scratch_types=_agg_scratch(),
    )
    def sc_agg1(idxp_hbm, taba_hbm, tabb_hbm, zeros_hbm,
                outa_hbm, outb_hbm,
                ibuf0, ibuf1, gbuf0, gbuf1, acc, is0, is1, gs0, gs1):
        c = lax.axis_index("c")
        s = lax.axis_index("s")
        row0 = s * ROWS_PER_SUB
        ibufs, gbufs = (ibuf0, ibuf1), (gbuf0, gbuf1)
        isems, gsems = (is0, is1), (gs0, gs1)

        pltpu.sync_copy(zeros_hbm.at[pl.ds(0, ROWS_PER_SUB)],
                        acc.at[pl.ds(row0, ROWS_PER_SUB)])
        plsc.subcore_barrier()

        def run(tab, outh):
            _agg_pipeline(idxp_hbm, s, tab, acc, ibufs, gbufs, isems,
                          gsems, NCH1)
            plsc.subcore_barrier()
            pltpu.sync_copy(acc.at[pl.ds(row0, ROWS_PER_SUB)],
                            outh.at[pl.ds(row0, ROWS_PER_SUB)])

        @pl.when(c == 0)
        def _():
            run(taba_hbm, outa_hbm)

        @pl.when(c == 1)
        def _():
            run(tabb_hbm, outb_hbm)

    return sc_agg1


def _make_sc_agg2(mesh):
    @functools.partial(
        pl.kernel,
        mesh=mesh,
        out_type=[
            jax.ShapeDtypeStruct((NPAD, D_IN), jnp.float32),
            jax.ShapeDtypeStruct((NPAD, D_IN), jnp.float32),
        ],
        compiler_params=_SC_PARAMS,
        scratch_types=_agg_scratch(),
    )
    def sc_agg2(idxp_hbm, tab_hbm, zeros_hbm, outa_hbm, outb_hbm,
                ibuf0, ibuf1, gbuf0, gbuf1, acc, is0, is1, gs0, gs1):
        c = lax.axis_index("c")
        s = lax.axis_index("s")
        w = s * NCORE + c
        row0 = s * ROWS_PER_SUB
        ibufs, gbufs = (ibuf0, ibuf1), (gbuf0, gbuf1)
        isems, gsems = (is0, is1), (gs0, gs1)

        pltpu.sync_copy(zeros_hbm.at[pl.ds(0, ROWS_PER_SUB)],
                        acc.at[pl.ds(row0, ROWS_PER_SUB)])
        plsc.subcore_barrier()

        _agg_pipeline(idxp_hbm, w, tab_hbm, acc, ibufs, gbufs, isems,
                      gsems, NCH2)
        plsc.subcore_barrier()

        @pl.when(c == 0)
        def _():
            pltpu.sync_copy(acc.at[pl.ds(row0, ROWS_PER_SUB)],
                            outa_hbm.at[pl.ds(row0, ROWS_PER_SUB)])

        @pl.when(c == 1)
        def _():
            pltpu.sync_copy(acc.at[pl.ds(row0, ROWS_PER_SUB)],
                            outb_hbm.at[pl.ds(row0, ROWS_PER_SUB)])

    return sc_agg2


_SC_CACHE = {}


def _sc_kernels():
    if not _SC_CACHE:
        mesh = plsc.VectorSubcoreMesh(core_axis_name="c", subcore_axis_name="s",
                                      num_cores=NCORE, num_subcores=NSUB)
        _SC_CACHE["deg"] = _make_sc_deg(mesh)
        _SC_CACHE["agg1"] = _make_sc_agg1(mesh)
        _SC_CACHE["agg2"] = _make_sc_agg2(mesh)
    return _SC_CACHE


def _dinv_of(degp):
    return lax.rsqrt(jnp.sum(degp, axis=0) + 1.0)


def _tc1_body(x_ref, w1_ref, degp_ref, ha_ref, hb_ref):
    dinv = _dinv_of(degp_ref[...])
    h = jnp.dot(x_ref[...], w1_ref[...], preferred_element_type=jnp.float32)
    hs = h * dinv[:, None]
    ha_ref[...] = hs[:, : D_H // 2]
    hb_ref[...] = hs[:, D_H // 2:]


def _tc2_body(a1a_ref, a1b_ref, h1a_ref, h1b_ref, degp_ref, b1_ref, w2_ref,
              o_ref):
    dinv = _dinv_of(degp_ref[...])
    b1 = b1_ref[...]
    za = jax.nn.relu((a1a_ref[...] + h1a_ref[...]) * dinv[:, None]
                     + b1[None, : D_H // 2])
    zb = jax.nn.relu((a1b_ref[...] + h1b_ref[...]) * dinv[:, None]
                     + b1[None, D_H // 2:])
    h2 = (jnp.dot(za, w2_ref[: D_H // 2, :],
                  preferred_element_type=jnp.float32)
          + jnp.dot(zb, w2_ref[D_H // 2:, :],
                    preferred_element_type=jnp.float32))
    o_ref[...] = h2 * dinv[:, None]


def _tc3_body(a2p0_ref, a2p1_ref, h2s_ref, degp_ref, b2_ref, o_ref):
    dinv = _dinv_of(degp_ref[...])
    o_ref[...] = ((a2p0_ref[...] + a2p1_ref[...] + h2s_ref[...])
                  * dinv[:, None] + b2_ref[...][None, :])


def _rows(shape_cols):
    return pl.BlockSpec((ROWB, shape_cols), lambda i: (i, 0))


def _full(shape):
    return pl.BlockSpec(shape, lambda i: tuple(0 for _ in shape))


_DEGP_SPEC = pl.BlockSpec((DEG_W, ROWB), lambda i: (0, i))

_tc1 = pl.pallas_call(
    _tc1_body,
    grid=(GRID,),
    in_specs=[_rows(D_IN), _full((D_IN, D_H)), _DEGP_SPEC],
    out_specs=[_rows(D_H // 2), _rows(D_H // 2)],
    out_shape=[
        jax.ShapeDtypeStruct((NPAD, D_H // 2), jnp.float32),
        jax.ShapeDtypeStruct((NPAD, D_H // 2), jnp.float32),
    ],
)

_tc2 = pl.pallas_call(
    _tc2_body,
    grid=(GRID,),
    in_specs=[_rows(D_H // 2)] * 4 + [_DEGP_SPEC, _full((D_H,)),
                                      _full((D_H, D_LAT))],
    out_specs=_rows(D_LAT),
    out_shape=jax.ShapeDtypeStruct((NPAD, D_LAT), jnp.float32),
)

_tc3 = pl.pallas_call(
    _tc3_body,
    grid=(GRID,),
    in_specs=[_rows(D_LAT), _rows(D_LAT), _rows(D_LAT), _DEGP_SPEC,
              _full((D_LAT,))],
    out_specs=_rows(D_LAT),
    out_shape=jax.ShapeDtypeStruct((NPAD, D_LAT), jnp.float32),
)


def _pack_idx(src, dst, groups, nch):
    pad = groups * nch * CH - E
    dummy = N + jnp.arange(pad, dtype=jnp.int32) % (NPAD - N)
    srcp = jnp.concatenate([src, jnp.zeros((pad,), jnp.int32)])
    dstp = jnp.concatenate([dst, dummy])
    return jnp.stack([srcp.reshape(groups, nch, CH),
                      dstp.reshape(groups, nch, CH)], axis=2)


def kernel(x, edge_index, W1, b1, W2, b2):
    src = edge_index[0]
    dst = edge_index[1]

    sc = _sc_kernels()

    degp = sc["deg"](dst.reshape(DEG_W, DEG_PER_W))

    idx1 = _pack_idx(src, dst, NSUB, NCH1)
    idx2 = _pack_idx(src, dst, DEG_W, NCH2)

    xp = jnp.zeros((NPAD, D_IN), jnp.float32).at[:N].set(x)
    zeros_acc = jnp.zeros((NPAD, D_IN), jnp.float32)

    h1a, h1b = _tc1(xp, W1, degp)
    a1a, a1b = sc["agg1"](idx1, h1a, h1b, zeros_acc)
    h2s = _tc2(a1a, a1b, h1a, h1b, degp, b1, W2)
    a2p0, a2p1 = sc["agg2"](idx2, h2s, zeros_acc)
    out = _tc3(a2p0, a2p1, h2s, degp, b2)
    return out[:N]


# speedup vs baseline: 25.8104x; 2.1404x over previous
import functools

import jax
import jax.numpy as jnp
from jax import lax
from jax.experimental import pallas as pl
from jax.experimental.pallas import tpu as pltpu
from jax.experimental.pallas import tpu_sc as plsc

N = 10000
E = 320000
D_IN = 128
D_H = 256
D_LAT = 128

NPAD = 10240
ROWB = 1024
GRID = NPAD // ROWB

NSUB = 16
NCORE = 2
CH = 128
NCH1 = 158
NCH2 = 80
ROWS_PER_SUB = NPAD // NSUB
DEG_W = NCORE * NSUB
DEG_PER_W = E // DEG_W

_SC_PARAMS = pltpu.CompilerParams(needs_layout_passes=False)


def _make_sc_deg(mesh):
    @functools.partial(
        pl.kernel,
        mesh=mesh,
        out_type=jax.ShapeDtypeStruct((DEG_W, NPAD), jnp.float32),
        compiler_params=_SC_PARAMS,
        scratch_types=[
            pltpu.VMEM((DEG_PER_W,), jnp.int32),
            pltpu.VMEM((NPAD,), jnp.float32),
        ],
    )
    def sc_deg(dst_hbm, out_hbm, dstv, acc):
        c = lax.axis_index("c")
        s = lax.axis_index("s")
        w = s * NCORE + c
        pltpu.sync_copy(dst_hbm.at[w], dstv)

        def zero_body(i, carry):
            acc[pl.ds(i * 16, 16)] = jnp.zeros((16,), jnp.float32)
            return carry

        lax.fori_loop(0, NPAD // 16, zero_body, None)

        ones = jnp.full((16,), 1.0, jnp.float32)

        def body(i, carry):
            idx = dstv[pl.ds(i * 16, 16)]
            plsc.addupdate_scatter(acc, [idx], ones)
            return carry

        lax.fori_loop(0, DEG_PER_W // 16, body, None)
        pltpu.sync_copy(acc, out_hbm.at[w])

    return sc_deg


def _agg_pipeline(idxp, g, tab, acc, ibufs, gbufs, isems, gsems, nch):
    for b in range(2):
        pltpu.async_copy(idxp.at[g, b], ibufs[b], isems[b])
    pltpu.make_async_copy(idxp.at[g, 0], ibufs[0], isems[0]).wait()
    pltpu.async_copy(tab.at[ibufs[0].at[0]], gbufs[0], gsems[0])

    def body(i, carry):
        for b in range(2):
            cc = 2 * i + b
            nb = 1 - b

            @pl.when(cc + 1 < nch)
            def _():
                pltpu.make_async_copy(idxp.at[g, cc + 1], ibufs[nb],
                                      isems[nb]).wait()
                pltpu.async_copy(tab.at[ibufs[nb].at[0]], gbufs[nb],
                                 gsems[nb])

            pltpu.make_async_copy(tab.at[ibufs[b].at[0]], gbufs[b],
                                  gsems[b]).wait()
            pltpu.sync_copy(gbufs[b], acc.at[ibufs[b].at[1]], add=True)

            @pl.when(cc + 2 < nch)
            def _():
                pltpu.async_copy(idxp.at[g, cc + 2], ibufs[b], isems[b])

        return carry

    lax.fori_loop(0, nch // 2, body, None)


def _agg_scratch():
    return [
        pltpu.VMEM((2, CH), jnp.int32),
        pltpu.VMEM((2, CH), jnp.int32),
        pltpu.VMEM((CH, D_IN), jnp.float32),
        pltpu.VMEM((CH, D_IN), jnp.float32),
        pltpu.VMEM_SHARED((NPAD, D_IN), jnp.float32),
        pltpu.SemaphoreType.DMA,
        pltpu.SemaphoreType.DMA,
        pltpu.SemaphoreType.DMA,
        pltpu.SemaphoreType.DMA,
    ]


def _make_sc_agg1(mesh):
    @functools.partial(
        pl.kernel,
        mesh=mesh,
        out_type=[
            jax.ShapeDtypeStruct((NPAD, D_IN), jnp.float32),
            jax.ShapeDtypeStruct((NPAD, D_IN), jnp.float32),
        ],
        compiler_params=_SC_PARAMS,
        scratch_types=_agg_scratch(),
    )
    def sc_agg1(idxp_hbm, taba_hbm, tabb_hbm, zeros_hbm,
                outa_hbm, outb_hbm,
                ibuf0, ibuf1, gbuf0, gbuf1, acc, is0, is1, gs0, gs1):
        c = lax.axis_index("c")
        s = lax.axis_index("s")
        row0 = s * ROWS_PER_SUB
        ibufs, gbufs = (ibuf0, ibuf1), (gbuf0, gbuf1)
        isems, gsems = (is0, is1), (gs0, gs1)

        pltpu.sync_copy(zeros_hbm.at[pl.ds(0, ROWS_PER_SUB)],
                        acc.at[pl.ds(row0, ROWS_PER_SUB)])
        plsc.subcore_barrier()

        def run(tab, outh):
            _agg_pipeline(idxp_hbm, s, tab, acc, ibufs, gbufs, isems,
                          gsems, NCH1)
            plsc.subcore_barrier()
            pltpu.sync_copy(acc.at[pl.ds(row0, ROWS_PER_SUB)],
                            outh.at[pl.ds(row0, ROWS_PER_SUB)])

        @pl.when(c == 0)
        def _():
            run(taba_hbm, outa_hbm)

        @pl.when(c == 1)
        def _():
            run(tabb_hbm, outb_hbm)

    return sc_agg1


def _make_sc_agg2(mesh):
    @functools.partial(
        pl.kernel,
        mesh=mesh,
        out_type=[
            jax.ShapeDtypeStruct((NPAD, D_IN), jnp.float32),
            jax.ShapeDtypeStruct((NPAD, D_IN), jnp.float32),
        ],
        compiler_params=_SC_PARAMS,
        scratch_types=_agg_scratch(),
    )
    def sc_agg2(idxp_hbm, tab_hbm, zeros_hbm, outa_hbm, outb_hbm,
                ibuf0, ibuf1, gbuf0, gbuf1, acc, is0, is1, gs0, gs1):
        c = lax.axis_index("c")
        s = lax.axis_index("s")
        w = s * NCORE + c
        row0 = s * ROWS_PER_SUB
        ibufs, gbufs = (ibuf0, ibuf1), (gbuf0, gbuf1)
        isems, gsems = (is0, is1), (gs0, gs1)

        pltpu.sync_copy(zeros_hbm.at[pl.ds(0, ROWS_PER_SUB)],
                        acc.at[pl.ds(row0, ROWS_PER_SUB)])
        plsc.subcore_barrier()

        _agg_pipeline(idxp_hbm, w, tab_hbm, acc, ibufs, gbufs, isems,
                      gsems, NCH2)
        plsc.subcore_barrier()

        @pl.when(c == 0)
        def _():
            pltpu.sync_copy(acc.at[pl.ds(row0, ROWS_PER_SUB)],
                            outa_hbm.at[pl.ds(row0, ROWS_PER_SUB)])

        @pl.when(c == 1)
        def _():
            pltpu.sync_copy(acc.at[pl.ds(row0, ROWS_PER_SUB)],
                            outb_hbm.at[pl.ds(row0, ROWS_PER_SUB)])

    return sc_agg2


_SC_CACHE = {}


def _sc_kernels():
    if not _SC_CACHE:
        mesh = plsc.VectorSubcoreMesh(core_axis_name="c", subcore_axis_name="s",
                                      num_cores=NCORE, num_subcores=NSUB)
        _SC_CACHE["deg"] = _make_sc_deg(mesh)
        _SC_CACHE["agg1"] = _make_sc_agg1(mesh)
        _SC_CACHE["agg2"] = _make_sc_agg2(mesh)
    return _SC_CACHE


def _dinv_of(degp):
    return lax.rsqrt(jnp.sum(degp, axis=0) + 1.0)


def _tc1_body(x_ref, w1_ref, degp_ref, ha_ref, hb_ref):
    dinv = _dinv_of(degp_ref[...])
    h = jnp.dot(x_ref[...], w1_ref[...], preferred_element_type=jnp.float32)
    hs = h * dinv[:, None]
    ha_ref[...] = hs[:, : D_H // 2]
    hb_ref[...] = hs[:, D_H // 2:]


def _tc2_body(a1a_ref, a1b_ref, h1a_ref, h1b_ref, degp_ref, b1_ref, w2_ref,
              o_ref):
    dinv = _dinv_of(degp_ref[...])
    b1 = b1_ref[...]
    za = jax.nn.relu((a1a_ref[...] + h1a_ref[...]) * dinv[:, None]
                     + b1[None, : D_H // 2])
    zb = jax.nn.relu((a1b_ref[...] + h1b_ref[...]) * dinv[:, None]
                     + b1[None, D_H // 2:])
    h2 = (jnp.dot(za, w2_ref[: D_H // 2, :],
                  preferred_element_type=jnp.float32)
          + jnp.dot(zb, w2_ref[D_H // 2:, :],
                    preferred_element_type=jnp.float32))
    o_ref[...] = h2 * dinv[:, None]


def _tc3_body(a2p0_ref, a2p1_ref, h2s_ref, degp_ref, b2_ref, o_ref):
    dinv = _dinv_of(degp_ref[...])
    o_ref[...] = ((a2p0_ref[...] + a2p1_ref[...] + h2s_ref[...])
                  * dinv[:, None] + b2_ref[...][None, :])


def _rows(shape_cols):
    return pl.BlockSpec((ROWB, shape_cols), lambda i: (i, 0))


def _full(shape):
    return pl.BlockSpec(shape, lambda i: tuple(0 for _ in shape))


_DEGP_SPEC = pl.BlockSpec((DEG_W, ROWB), lambda i: (0, i))

_tc1 = pl.pallas_call(
    _tc1_body,
    grid=(GRID,),
    in_specs=[_rows(D_IN), _full((D_IN, D_H)), _DEGP_SPEC],
    out_specs=[_rows(D_H // 2), _rows(D_H // 2)],
    out_shape=[
        jax.ShapeDtypeStruct((NPAD, D_H // 2), jnp.float32),
        jax.ShapeDtypeStruct((NPAD, D_H // 2), jnp.float32),
    ],
)

_tc2 = pl.pallas_call(
    _tc2_body,
    grid=(GRID,),
    in_specs=[_rows(D_H // 2)] * 4 + [_DEGP_SPEC, _full((D_H,)),
                                      _full((D_H, D_LAT))],
    out_specs=_rows(D_LAT),
    out_shape=jax.ShapeDtypeStruct((NPAD, D_LAT), jnp.float32),
)

_tc3 = pl.pallas_call(
    _tc3_body,
    grid=(GRID,),
    in_specs=[_rows(D_LAT), _rows(D_LAT), _rows(D_LAT), _DEGP_SPEC,
              _full((D_LAT,))],
    out_specs=_rows(D_LAT),
    out_shape=jax.ShapeDtypeStruct((NPAD, D_LAT), jnp.float32),
)


def _pack_idx(src, dst, groups, nch):
    pad = groups * nch * CH - E
    ar = jnp.arange(pad, dtype=jnp.int32)
    srcp = jnp.concatenate([src, ar % N])
    dstp = jnp.concatenate([dst, N + ar % (NPAD - N)])
    return jnp.stack([srcp.reshape(groups, nch, CH),
                      dstp.reshape(groups, nch, CH)], axis=2)


def kernel(x, edge_index, W1, b1, W2, b2):
    src = edge_index[0]
    dst = edge_index[1]

    sc = _sc_kernels()

    degp = sc["deg"](dst.reshape(DEG_W, DEG_PER_W))

    idx1 = _pack_idx(src, dst, NSUB, NCH1)
    idx2 = _pack_idx(src, dst, DEG_W, NCH2)

    xp = jnp.zeros((NPAD, D_IN), jnp.float32).at[:N].set(x)
    zeros_acc = jnp.zeros((NPAD, D_IN), jnp.float32)

    h1a, h1b = _tc1(xp, W1, degp)
    a1a, a1b = sc["agg1"](idx1, h1a, h1b, zeros_acc)
    h2s = _tc2(a1a, a1b, h1a, h1b, degp, b1, W2)
    a2p0, a2p1 = sc["agg2"](idx2, h2s, zeros_acc)
    out = _tc3(a2p0, a2p1, h2s, degp, b2)
    return out[:N]
